# baseline (speedup 1.0000x reference)
import functools

import jax
import jax.numpy as jnp
from jax import lax
from jax.experimental import pallas as pl
from jax.experimental.pallas import tpu as pltpu
from jax.experimental.pallas import tpu_sc as plsc

NC = 2
NS = 16
NW = NC * NS
L = 16
CHUNK = 128


@functools.lru_cache(maxsize=None)
def _make_seg_sum_ring(n_pad: int, width: int, qk: int, ch: int, nring: int,
                       nq: int):
    assert qk % nring == 0
    mesh = plsc.VectorSubcoreMesh(core_axis_name="c", subcore_axis_name="s")
    rows_per_sub = n_pad // NS

    @functools.partial(
        pl.kernel,
        out_type=jax.ShapeDtypeStruct((NC, n_pad, width), jnp.float32),
        mesh=mesh,
        scratch_types=(
            [
                pltpu.VMEM((qk, ch), jnp.int32),
                pltpu.VMEM((qk, ch), jnp.int32),
            ]
            + [pltpu.VMEM((ch, width), jnp.float32) for _ in range(nring)]
            + [pltpu.SemaphoreType.DMA for _ in range(nring)]
            + [pltpu.VMEM_SHARED((n_pad, width), jnp.float32)]
        ),
        compiler_params=pltpu.CompilerParams(use_tc_tiling_on_sc=False),
    )
    def seg(tbl_hbm, src_hbm, dst_hbm, out_hbm, src_q, dst_q, *bufs):
        rows = list(bufs[:nring])
        sems = list(bufs[nring:2 * nring])
        acc = bufs[2 * nring]
        cid = lax.axis_index("c")
        sid = lax.axis_index("s")
        base = sid * rows_per_sub
        zero = jnp.zeros((L,), jnp.float32)

        @pl.loop(0, ch)
        def _(i):
            for c in range(width // L):
                rows[0][i, pl.ds(c * L, L)] = zero

        zf = rows_per_sub // ch
        zr = rows_per_sub % ch

        @pl.loop(0, zf)
        def _(z):
            pltpu.sync_copy(rows[0], acc.at[pl.ds(base + z * ch, ch)])

        if zr:
            pltpu.sync_copy(rows[0].at[pl.ds(0, zr)],
                            acc.at[pl.ds(base + zf * ch, zr)])

        plsc.subcore_barrier()

        for p in range(nq):
            pltpu.sync_copy(src_hbm.at[cid, sid, p], src_q)
            pltpu.sync_copy(dst_hbm.at[cid, sid, p], dst_q)
            for b in range(nring):
                pltpu.async_copy(tbl_hbm.at[src_q.at[b]], rows[b], sems[b])

            @pl.loop(0, qk // nring - 1)
            def _(g):
                for b in range(nring):
                    q = g * nring + b
                    pltpu.make_async_copy(tbl_hbm.at[src_q.at[q]], rows[b],
                                          sems[b]).wait()
                    pltpu.sync_copy(rows[b], acc.at[dst_q.at[q]], add=True)
                    pltpu.async_copy(tbl_hbm.at[src_q.at[q + nring]], rows[b],
                                     sems[b])

            for b in range(nring):
                q = qk - nring + b
                pltpu.make_async_copy(tbl_hbm.at[src_q.at[q]], rows[b],
                                      sems[b]).wait()
                pltpu.sync_copy(rows[b], acc.at[dst_q.at[q]], add=True)

        plsc.subcore_barrier()

        @pl.loop(0, zf)
        def _(z):
            sl = pl.ds(base + z * ch, ch)
            pltpu.sync_copy(acc.at[sl], rows[0])
            pltpu.sync_copy(rows[0], out_hbm.at[cid, sl])

        if zr:
            sl = pl.ds(base + zf * ch, zr)
            pltpu.sync_copy(acc.at[sl], rows[0].at[pl.ds(0, zr)])
            pltpu.sync_copy(rows[0].at[pl.ds(0, zr)], out_hbm.at[cid, sl])

    return seg


@functools.lru_cache(maxsize=None)
def _make_seg_sum_split(n_pad: int, width: int, qk: int, ch: int, nring: int,
                        nq: int):
    assert qk % nring == 0
    mesh = plsc.VectorSubcoreMesh(core_axis_name="c", subcore_axis_name="s")
    rows_per_sub = n_pad // NS

    @functools.partial(
        pl.kernel,
        out_type=jax.ShapeDtypeStruct((NC, n_pad, width), jnp.float32),
        mesh=mesh,
        scratch_types=(
            [
                pltpu.VMEM((qk, ch), jnp.int32),
                pltpu.VMEM((qk, ch), jnp.int32),
            ]
            + [pltpu.VMEM((ch, width), jnp.float32) for _ in range(nring)]
            + [pltpu.SemaphoreType.DMA for _ in range(nring)]
            + [pltpu.VMEM_SHARED((n_pad, width), jnp.float32)]
        ),
        compiler_params=pltpu.CompilerParams(use_tc_tiling_on_sc=False),
    )
    def seg(tblA_hbm, tblB_hbm, src_hbm, dst_hbm, out_hbm, src_q, dst_q,
            *bufs):
        rows = list(bufs[:nring])
        sems = list(bufs[nring:2 * nring])
        acc = bufs[2 * nring]
        cid = lax.axis_index("c")
        sid = lax.axis_index("s")
        base = sid * rows_per_sub
        zero = jnp.zeros((L,), jnp.float32)

        @pl.loop(0, ch)
        def _(i):
            for c in range(width // L):
                rows[0][i, pl.ds(c * L, L)] = zero

        zf = rows_per_sub // ch
        zr = rows_per_sub % ch

        @pl.loop(0, zf)
        def _(z):
            pltpu.sync_copy(rows[0], acc.at[pl.ds(base + z * ch, ch)])

        if zr:
            pltpu.sync_copy(rows[0].at[pl.ds(0, zr)],
                            acc.at[pl.ds(base + zf * ch, zr)])

        plsc.subcore_barrier()

        for p in range(nq):
            pltpu.sync_copy(src_hbm.at[sid, p], src_q)
            pltpu.sync_copy(dst_hbm.at[sid, p], dst_q)

            def run(tbl_hbm):
                for b in range(nring):
                    pltpu.async_copy(tbl_hbm.at[src_q.at[b]], rows[b],
                                     sems[b])

                @pl.loop(0, qk // nring - 1)
                def _(g):
                    for b in range(nring):
                        q = g * nring + b
                        pltpu.make_async_copy(tbl_hbm.at[src_q.at[q]],
                                              rows[b], sems[b]).wait()
                        pltpu.sync_copy(rows[b], acc.at[dst_q.at[q]],
                                        add=True)
                        pltpu.async_copy(tbl_hbm.at[src_q.at[q + nring]],
                                         rows[b], sems[b])

                for b in range(nring):
                    q = qk - nring + b
                    pltpu.make_async_copy(tbl_hbm.at[src_q.at[q]], rows[b],
                                          sems[b]).wait()
                    pltpu.sync_copy(rows[b], acc.at[dst_q.at[q]], add=True)

            @pl.when(cid == 0)
            def _():
                run(tblA_hbm)

            @pl.when(cid == 1)
            def _():
                run(tblB_hbm)

        plsc.subcore_barrier()

        @pl.loop(0, zf)
        def _(z):
            sl = pl.ds(base + z * ch, ch)
            pltpu.sync_copy(acc.at[sl], rows[0])
            pltpu.sync_copy(rows[0], out_hbm.at[cid, sl])

        if zr:
            sl = pl.ds(base + zf * ch, zr)
            pltpu.sync_copy(acc.at[sl], rows[0].at[pl.ds(0, zr)])
            pltpu.sync_copy(rows[0].at[pl.ds(0, zr)], out_hbm.at[cid, sl])

    return seg


_DN = (((1,), (1,)), ((), ()))


def _proj1(x, Wl1, Wr1, bn):
    n, d = x.shape
    h = Wl1.shape[0]

    hh = h // 2
    ws = hh + L

    def body(x_ref, wl_ref, wr_ref, ta_ref, tb_ref, xr_ref):
        xb = x_ref[...]
        xl = lax.dot_general(xb, wl_ref[...], _DN,
                             preferred_element_type=jnp.float32)
        tail = (lax.broadcasted_iota(jnp.int32, (bn, L), 1) == 0)
        tail = tail.astype(jnp.float32)
        ta_ref[...] = jnp.concatenate([xl[:, :hh], tail], axis=1)
        tb_ref[...] = jnp.concatenate([xl[:, hh:], tail], axis=1)
        xr_ref[...] = lax.dot_general(xb, wr_ref[...], _DN,
                                      preferred_element_type=jnp.float32)

    return pl.pallas_call(
        body,
        grid=(n // bn,),
        in_specs=[
            pl.BlockSpec((bn, d), lambda i: (i, 0)),
            pl.BlockSpec((h, d), lambda i: (0, 0)),
            pl.BlockSpec((h, d), lambda i: (0, 0)),
        ],
        out_specs=[
            pl.BlockSpec((bn, ws), lambda i: (i, 0)),
            pl.BlockSpec((bn, ws), lambda i: (i, 0)),
            pl.BlockSpec((bn, h), lambda i: (i, 0)),
        ],
        out_shape=[
            jax.ShapeDtypeStruct((n, ws), jnp.float32),
            jax.ShapeDtypeStruct((n, ws), jnp.float32),
            jax.ShapeDtypeStruct((n, h), jnp.float32),
        ],
    )(x, Wl1, Wr1)


def _mid(acc1, xr, b1, W16, bb, bn):
    n, h = xr.shape
    w1 = acc1.shape[2]

    hh = h // 2

    def body(acc_ref, xr_ref, b1_ref, w16_ref, bb_ref, tbl_ref):
        p = jnp.concatenate([acc_ref[0, :, :hh], acc_ref[1, :, :hh]], axis=1)
        deg = jnp.maximum(acc_ref[0, :, hh:hh + 1], 1.0)
        hid = jnp.maximum(p / deg + b1_ref[...] + xr_ref[...], 0.0)
        t = lax.dot_general(hid, w16_ref[...], _DN,
                            preferred_element_type=jnp.float32)
        col = lax.broadcasted_iota(jnp.int32, (bn, L), 1)
        tbl_ref[...] = t + bb_ref[...] + deg * (col == 4).astype(jnp.float32)

    return pl.pallas_call(
        body,
        grid=(n // bn,),
        in_specs=[
            pl.BlockSpec((NC, bn, w1), lambda i: (0, i, 0)),
            pl.BlockSpec((bn, h), lambda i: (i, 0)),
            pl.BlockSpec((1, h), lambda i: (0, 0)),
            pl.BlockSpec((L, h), lambda i: (0, 0)),
            pl.BlockSpec((1, L), lambda i: (0, 0)),
        ],
        out_specs=pl.BlockSpec((bn, L), lambda i: (i, 0)),
        out_shape=jax.ShapeDtypeStruct((n, L), jnp.float32),
    )(acc1, xr, b1, W16, bb)


def _final(acc2, tbl2, o, bn):
    n = tbl2.shape[0]

    def body(acc_ref, tbl_ref, out_ref):
        s = acc_ref[0] + acc_ref[1]
        deg = tbl_ref[:, 4:5]
        y = s[:, :o] / deg + tbl_ref[:, 8:8 + o]
        m = jnp.max(y, axis=1, keepdims=True)
        e = jnp.exp(y - m)
        out_ref[...] = e / jnp.sum(e, axis=1, keepdims=True)

    return pl.pallas_call(
        body,
        grid=(n // bn,),
        in_specs=[
            pl.BlockSpec((NC, bn, L), lambda i: (0, i, 0)),
            pl.BlockSpec((bn, L), lambda i: (i, 0)),
        ],
        out_specs=pl.BlockSpec((bn, o), lambda i: (i, 0)),
        out_shape=jax.ShapeDtypeStruct((n, o), jnp.float32),
    )(acc2, tbl2)


def kernel(x, edge_index, Wl1, Wr1, b1, Wl2, Wr2, b2):
    n, d = x.shape
    e = edge_index.shape[1]
    h = Wl1.shape[0]
    o = Wl2.shape[0]

    k = -(-e // (NW * CHUNK))
    k += k % 2
    e_pad = k * NW * CHUNK
    ep_tec = e_pad // NW
    n_pad = -(-(n + 1) // NS) * NS
    bn = 1000 if n % 1000 == 0 else 8

    src = edge_index[0]
    dst = edge_index[1]
    pad = e_pad - e
    ar = jnp.arange(pad, dtype=jnp.int32)
    srcp = jnp.concatenate([src, (ar * 37) % n])
    dstp = jnp.concatenate([dst, n + ar % (n_pad - n)])
    srcp = srcp.reshape(NC, NS, ep_tec)
    dstp = dstp.reshape(NC, NS, ep_tec)

    ch1 = CHUNK // 2
    k1 = (e_pad // NS) // ch1
    nq1, nring1 = 8, 8
    qk1 = k1 // nq1
    tblA, tblB, xr = _proj1(x, Wl1, Wr1, bn)
    acc1 = _make_seg_sum_split(n_pad, h // 2 + L, qk1, ch1, nring1, nq1)(
        tblA, tblB, srcp.reshape(NS, nq1, qk1, ch1),
        dstp.reshape(NS, nq1, qk1, ch1))

    W16 = jnp.zeros((L, h), jnp.float32).at[0:o].set(Wl2).at[8:8 + o].set(Wr2)
    bb = jnp.zeros((1, L), jnp.float32).at[0, 8:8 + o].set(b2)
    tbl2 = _mid(acc1, xr, b1.reshape(1, h), W16, bb, bn)

    acc2 = _make_seg_sum_ring(n_pad, L, k, CHUNK, 8, 1)(
        tbl2, srcp.reshape(NC, NS, 1, k, CHUNK),
        dstp.reshape(NC, NS, 1, k, CHUNK))
    return _final(acc2, tbl2, o, bn)


# speedup vs baseline: 1.0100x; 1.0100x over previous
import functools

import jax
import jax.numpy as jnp
from jax import lax
from jax.experimental import pallas as pl
from jax.experimental.pallas import tpu as pltpu
from jax.experimental.pallas import tpu_sc as plsc

NC = 2
NS = 16
NW = NC * NS
L = 16
CHUNK = 128


@functools.lru_cache(maxsize=None)
def _make_seg_sum_ring(n_pad: int, width: int, qk: int, ch: int, nring: int,
                       nq: int):
    assert qk % nring == 0
    mesh = plsc.VectorSubcoreMesh(core_axis_name="c", subcore_axis_name="s")
    rows_per_sub = n_pad // NS

    @functools.partial(
        pl.kernel,
        out_type=jax.ShapeDtypeStruct((NC, n_pad, width), jnp.float32),
        mesh=mesh,
        scratch_types=(
            [
                pltpu.VMEM((qk, ch), jnp.int32),
                pltpu.VMEM((qk, ch), jnp.int32),
            ]
            + [pltpu.VMEM((ch, width), jnp.float32) for _ in range(nring)]
            + [pltpu.SemaphoreType.DMA for _ in range(nring)]
            + [pltpu.VMEM_SHARED((n_pad, width), jnp.float32)]
        ),
        compiler_params=pltpu.CompilerParams(use_tc_tiling_on_sc=False),
    )
    def seg(tbl_hbm, src_hbm, dst_hbm, out_hbm, src_q, dst_q, *bufs):
        rows = list(bufs[:nring])
        sems = list(bufs[nring:2 * nring])
        acc = bufs[2 * nring]
        cid = lax.axis_index("c")
        sid = lax.axis_index("s")
        base = sid * rows_per_sub
        zero = jnp.zeros((L,), jnp.float32)

        @pl.loop(0, ch)
        def _(i):
            for c in range(width // L):
                rows[0][i, pl.ds(c * L, L)] = zero

        zf = rows_per_sub // ch
        zr = rows_per_sub % ch

        @pl.loop(0, zf)
        def _(z):
            pltpu.sync_copy(rows[0], acc.at[pl.ds(base + z * ch, ch)])

        if zr:
            pltpu.sync_copy(rows[0].at[pl.ds(0, zr)],
                            acc.at[pl.ds(base + zf * ch, zr)])

        plsc.subcore_barrier()

        for p in range(nq):
            pltpu.sync_copy(src_hbm.at[cid, sid, p], src_q)
            pltpu.sync_copy(dst_hbm.at[cid, sid, p], dst_q)
            for b in range(nring):
                pltpu.async_copy(tbl_hbm.at[src_q.at[b]], rows[b], sems[b])

            @pl.loop(0, qk // nring - 1)
            def _(g):
                for b in range(nring):
                    q = g * nring + b
                    pltpu.make_async_copy(tbl_hbm.at[src_q.at[q]], rows[b],
                                          sems[b]).wait()
                    pltpu.sync_copy(rows[b], acc.at[dst_q.at[q]], add=True)
                    pltpu.async_copy(tbl_hbm.at[src_q.at[q + nring]], rows[b],
                                     sems[b])

            for b in range(nring):
                q = qk - nring + b
                pltpu.make_async_copy(tbl_hbm.at[src_q.at[q]], rows[b],
                                      sems[b]).wait()
                pltpu.sync_copy(rows[b], acc.at[dst_q.at[q]], add=True)

        plsc.subcore_barrier()

        @pl.loop(0, zf)
        def _(z):
            sl = pl.ds(base + z * ch, ch)
            pltpu.sync_copy(acc.at[sl], rows[0])
            pltpu.sync_copy(rows[0], out_hbm.at[cid, sl])

        if zr:
            sl = pl.ds(base + zf * ch, zr)
            pltpu.sync_copy(acc.at[sl], rows[0].at[pl.ds(0, zr)])
            pltpu.sync_copy(rows[0].at[pl.ds(0, zr)], out_hbm.at[cid, sl])

    return seg


_DN = (((1,), (1,)), ((), ()))


def _proj1(x, Wl1, Wr1, bn):
    n, d = x.shape
    h = Wl1.shape[0]

    w1 = h + L

    def body(x_ref, wl_ref, wr_ref, aug_ref, xr_ref):
        xb = x_ref[...]
        xl = lax.dot_general(xb, wl_ref[...], _DN,
                             preferred_element_type=jnp.float32)
        tail = (lax.broadcasted_iota(jnp.int32, (bn, L), 1) == 0)
        aug_ref[...] = jnp.concatenate([xl, tail.astype(jnp.float32)], axis=1)
        xr_ref[...] = lax.dot_general(xb, wr_ref[...], _DN,
                                      preferred_element_type=jnp.float32)

    return pl.pallas_call(
        body,
        grid=(n // bn,),
        in_specs=[
            pl.BlockSpec((bn, d), lambda i: (i, 0)),
            pl.BlockSpec((h, d), lambda i: (0, 0)),
            pl.BlockSpec((h, d), lambda i: (0, 0)),
        ],
        out_specs=[
            pl.BlockSpec((bn, w1), lambda i: (i, 0)),
            pl.BlockSpec((bn, h), lambda i: (i, 0)),
        ],
        out_shape=[
            jax.ShapeDtypeStruct((n, w1), jnp.float32),
            jax.ShapeDtypeStruct((n, h), jnp.float32),
        ],
    )(x, Wl1, Wr1)


def _mid(acc1, xr, b1, W16, bb, bn):
    n, h = xr.shape
    w1 = acc1.shape[2]

    def body(acc_ref, xr_ref, b1_ref, w16_ref, bb_ref, tbl_ref):
        p = acc_ref[0] + acc_ref[1]
        deg = jnp.maximum(p[:, h:h + 1], 1.0)
        hid = jnp.maximum(p[:, :h] / deg + b1_ref[...] + xr_ref[...], 0.0)
        t = lax.dot_general(hid, w16_ref[...], _DN,
                            preferred_element_type=jnp.float32)
        col = lax.broadcasted_iota(jnp.int32, (bn, L), 1)
        tbl_ref[...] = t + bb_ref[...] + deg * (col == 4).astype(jnp.float32)

    return pl.pallas_call(
        body,
        grid=(n // bn,),
        in_specs=[
            pl.BlockSpec((NC, bn, w1), lambda i: (0, i, 0)),
            pl.BlockSpec((bn, h), lambda i: (i, 0)),
            pl.BlockSpec((1, h), lambda i: (0, 0)),
            pl.BlockSpec((L, h), lambda i: (0, 0)),
            pl.BlockSpec((1, L), lambda i: (0, 0)),
        ],
        out_specs=pl.BlockSpec((bn, L), lambda i: (i, 0)),
        out_shape=jax.ShapeDtypeStruct((n, L), jnp.float32),
    )(acc1, xr, b1, W16, bb)


def _final(acc2, tbl2, o, bn):
    n = tbl2.shape[0]

    def body(acc_ref, tbl_ref, out_ref):
        s = acc_ref[0] + acc_ref[1]
        deg = tbl_ref[:, 4:5]
        y = s[:, :o] / deg + tbl_ref[:, 8:8 + o]
        m = jnp.max(y, axis=1, keepdims=True)
        e = jnp.exp(y - m)
        out_ref[...] = e / jnp.sum(e, axis=1, keepdims=True)

    return pl.pallas_call(
        body,
        grid=(n // bn,),
        in_specs=[
            pl.BlockSpec((NC, bn, L), lambda i: (0, i, 0)),
            pl.BlockSpec((bn, L), lambda i: (i, 0)),
        ],
        out_specs=pl.BlockSpec((bn, o), lambda i: (i, 0)),
        out_shape=jax.ShapeDtypeStruct((n, o), jnp.float32),
    )(acc2, tbl2)


def kernel(x, edge_index, Wl1, Wr1, b1, Wl2, Wr2, b2):
    n, d = x.shape
    e = edge_index.shape[1]
    h = Wl1.shape[0]
    o = Wl2.shape[0]

    k = -(-e // (NW * CHUNK))
    k += k % 2
    e_pad = k * NW * CHUNK
    ep_tec = e_pad // NW
    n_pad = -(-(n + 1) // NS) * NS
    bn = 1000 if n % 1000 == 0 else 8

    src = edge_index[0]
    dst = edge_index[1]
    pad = e_pad - e
    ar = jnp.arange(pad, dtype=jnp.int32)
    srcp = jnp.concatenate([src, (ar * 37) % n])
    dstp = jnp.concatenate([dst, n + ar % (n_pad - n)])
    srcp = srcp.reshape(NC, NS, ep_tec)
    dstp = dstp.reshape(NC, NS, ep_tec)

    ch1 = CHUNK // 2
    k1 = ep_tec // ch1
    nq1, nring1 = 8, 4
    qk1 = k1 // nq1
    aug, xr = _proj1(x, Wl1, Wr1, bn)
    acc1 = _make_seg_sum_ring(n_pad, h + L, qk1, ch1, nring1, nq1)(
        aug, srcp.reshape(NC, NS, nq1, qk1, ch1),
        dstp.reshape(NC, NS, nq1, qk1, ch1))

    W16 = jnp.zeros((L, h), jnp.float32).at[0:o].set(Wl2).at[8:8 + o].set(Wr2)
    bb = jnp.zeros((1, L), jnp.float32).at[0, 8:8 + o].set(b2)
    tbl2 = _mid(acc1, xr, b1.reshape(1, h), W16, bb, bn)

    acc2 = _make_seg_sum_ring(n_pad, L, k, CHUNK, 8, 1)(
        tbl2, srcp.reshape(NC, NS, 1, k, CHUNK),
        dstp.reshape(NC, NS, 1, k, CHUNK))
    return _final(acc2, tbl2, o, bn)
